# baseline (speedup 1.0000x reference)
import jax
import jax.numpy as jnp
from jax.experimental import pallas as pl
from jax.experimental.pallas import tpu as pltpu

B, V, D_IN, D_H, D_OUT = 1024, 128, 256, 512, 256
VT = 4


def _mlp_kernel(x_ref, w0_ref, b0_ref, w1_ref, b1_ref, w2_ref, b2_ref, out_ref):
    bf = jnp.bfloat16
    dn = (((1,), (1,)), ((), ()))
    for i in range(VT):
        xv = x_ref[:, i, 0, :]
        h = jax.lax.dot_general(xv, w0_ref[i], dn, preferred_element_type=jnp.float32)
        h = jnp.maximum(h + b0_ref[i], 0.0).astype(bf)
        h = jax.lax.dot_general(h, w1_ref[i], dn, preferred_element_type=jnp.float32)
        h = jnp.maximum(h + b1_ref[i], 0.0).astype(bf)
        o = jax.lax.dot_general(h, w2_ref[i], dn, preferred_element_type=jnp.float32)
        out_ref[:, i, 0, :] = o + b2_ref[i]


def kernel(x, W0, b0, W1, b1, W2, b2):
    out = pl.pallas_call(
        _mlp_kernel,
        grid=(V // VT,),
        in_specs=[
            pl.BlockSpec((B, VT, 1, D_IN), lambda v: (0, v, 0, 0)),
            pl.BlockSpec((VT, D_H, D_IN), lambda v: (v, 0, 0)),
            pl.BlockSpec((VT, 1, D_H), lambda v: (v, 0, 0)),
            pl.BlockSpec((VT, D_H, D_H), lambda v: (v, 0, 0)),
            pl.BlockSpec((VT, 1, D_H), lambda v: (v, 0, 0)),
            pl.BlockSpec((VT, D_OUT, D_H), lambda v: (v, 0, 0)),
            pl.BlockSpec((VT, 1, D_OUT), lambda v: (v, 0, 0)),
        ],
        out_specs=pl.BlockSpec((B, VT, 1, D_OUT), lambda v: (0, v, 0, 0)),
        out_shape=jax.ShapeDtypeStruct((B, V, 1, D_OUT), jnp.float32),
        compiler_params=pltpu.CompilerParams(
            dimension_semantics=("parallel",),
            vmem_limit_bytes=120 * 1024 * 1024,
        ),
    )(
        x.reshape(B, V, 1, D_IN),
        W0,
        b0.reshape(V, 1, D_H),
        W1,
        b1.reshape(V, 1, D_H),
        W2,
        b2.reshape(V, 1, D_OUT),
    )
    return out.reshape(B, V, D_OUT)


# speedup vs baseline: 1.0047x; 1.0047x over previous
import jax
import jax.numpy as jnp
from jax.experimental import pallas as pl
from jax.experimental.pallas import tpu as pltpu

B, V, D_IN, D_H, D_OUT = 1024, 128, 256, 512, 256
VT = 4


def _mlp_kernel(x_ref, w0_ref, b0_ref, w1_ref, b1_ref, w2_ref, b2_ref, out_ref):
    v0 = pl.program_id(0) * VT
    dn = (((1,), (1,)), ((), ()))
    for i in range(VT):
        xv = x_ref[:, i, 0, :]
        h = jax.lax.dot_general(xv, w0_ref[i], dn, preferred_element_type=jnp.float32)
        h = jnp.maximum(h + b0_ref[v0 + i], 0.0)
        h = jax.lax.dot_general(h, w1_ref[i], dn, preferred_element_type=jnp.float32)
        h = jnp.maximum(h + b1_ref[v0 + i], 0.0)
        o = jax.lax.dot_general(h, w2_ref[i], dn, preferred_element_type=jnp.float32)
        out_ref[:, i, 0, :] = o + b2_ref[v0 + i]


def kernel(x, W0, b0, W1, b1, W2, b2):
    out = pl.pallas_call(
        _mlp_kernel,
        grid=(V // VT,),
        in_specs=[
            pl.BlockSpec((B, VT, 1, D_IN), lambda v: (0, v, 0, 0)),
            pl.BlockSpec((VT, D_H, D_IN), lambda v: (v, 0, 0)),
            pl.BlockSpec((V, 1, D_H), lambda v: (0, 0, 0)),
            pl.BlockSpec((VT, D_H, D_H), lambda v: (v, 0, 0)),
            pl.BlockSpec((V, 1, D_H), lambda v: (0, 0, 0)),
            pl.BlockSpec((VT, D_OUT, D_H), lambda v: (v, 0, 0)),
            pl.BlockSpec((V, 1, D_OUT), lambda v: (0, 0, 0)),
        ],
        out_specs=pl.BlockSpec((B, VT, 1, D_OUT), lambda v: (0, v, 0, 0)),
        out_shape=jax.ShapeDtypeStruct((B, V, 1, D_OUT), jnp.float32),
        compiler_params=pltpu.CompilerParams(
            dimension_semantics=("parallel",),
            vmem_limit_bytes=120 * 1024 * 1024,
        ),
    )(
        x.reshape(B, V, 1, D_IN),
        W0,
        b0.reshape(V, 1, D_H),
        W1,
        b1.reshape(V, 1, D_H),
        W2,
        b2.reshape(V, 1, D_OUT),
    )
    return out.reshape(B, V, D_OUT)


# speedup vs baseline: 1.0095x; 1.0048x over previous
import jax
import jax.numpy as jnp
from jax.experimental import pallas as pl
from jax.experimental.pallas import tpu as pltpu

B, V, D_IN, D_H, D_OUT = 1024, 128, 256, 512, 256
VT = 4
NSTEP = V // VT
NSLOT = 3


def _issue(w0_hbm, w1_hbm, w2_hbm, w0_s, w1_s, w2_s, sem, step):
    slot = step % NSLOT
    pltpu.make_async_copy(
        w0_hbm.at[pl.ds(step * VT, VT)], w0_s.at[slot], sem.at[slot, 0]).start()
    pltpu.make_async_copy(
        w1_hbm.at[pl.ds(step * VT, VT)], w1_s.at[slot], sem.at[slot, 1]).start()
    pltpu.make_async_copy(
        w2_hbm.at[pl.ds(step * VT, VT)], w2_s.at[slot], sem.at[slot, 2]).start()


def _wait(w0_s, w1_s, w2_s, sem, step):
    slot = step % NSLOT
    pltpu.make_async_copy(w0_s.at[slot], w0_s.at[slot], sem.at[slot, 0]).wait()
    pltpu.make_async_copy(w1_s.at[slot], w1_s.at[slot], sem.at[slot, 1]).wait()
    pltpu.make_async_copy(w2_s.at[slot], w2_s.at[slot], sem.at[slot, 2]).wait()


def _mlp_kernel(x_ref, b0_ref, b1_ref, b2_ref, w0_hbm, w1_hbm, w2_hbm,
                out_ref, w0_s, w1_s, w2_s, sem):
    v = pl.program_id(0)
    v0 = v * VT

    @pl.when(v == 0)
    def _prologue():
        for s in range(NSLOT):
            _issue(w0_hbm, w1_hbm, w2_hbm, w0_s, w1_s, w2_s, sem, s)

    _wait(w0_s, w1_s, w2_s, sem, v)
    slot = v % NSLOT

    dn = (((1,), (1,)), ((), ()))
    for i in range(VT):
        xv = x_ref[:, i, 0, :]
        h = jax.lax.dot_general(xv, w0_s[slot, i], dn,
                                preferred_element_type=jnp.float32)
        h = jnp.maximum(h + b0_ref[v0 + i], 0.0)
        h = jax.lax.dot_general(h, w1_s[slot, i], dn,
                                preferred_element_type=jnp.float32)
        h = jnp.maximum(h + b1_ref[v0 + i], 0.0)
        o = jax.lax.dot_general(h, w2_s[slot, i], dn,
                                preferred_element_type=jnp.float32)
        out_ref[:, i, 0, :] = o + b2_ref[v0 + i]

    @pl.when(v + NSLOT < NSTEP)
    def _prefetch():
        _issue(w0_hbm, w1_hbm, w2_hbm, w0_s, w1_s, w2_s, sem, v + NSLOT)


def kernel(x, W0, b0, W1, b1, W2, b2):
    out = pl.pallas_call(
        _mlp_kernel,
        grid=(NSTEP,),
        in_specs=[
            pl.BlockSpec((B, VT, 1, D_IN), lambda v: (0, v, 0, 0)),
            pl.BlockSpec((V, 1, D_H), lambda v: (0, 0, 0)),
            pl.BlockSpec((V, 1, D_H), lambda v: (0, 0, 0)),
            pl.BlockSpec((V, 1, D_OUT), lambda v: (0, 0, 0)),
            pl.BlockSpec(memory_space=pltpu.MemorySpace.HBM),
            pl.BlockSpec(memory_space=pltpu.MemorySpace.HBM),
            pl.BlockSpec(memory_space=pltpu.MemorySpace.HBM),
        ],
        out_specs=pl.BlockSpec((B, VT, 1, D_OUT), lambda v: (0, v, 0, 0)),
        out_shape=jax.ShapeDtypeStruct((B, V, 1, D_OUT), jnp.float32),
        scratch_shapes=[
            pltpu.VMEM((NSLOT, VT, D_H, D_IN), jnp.float32),
            pltpu.VMEM((NSLOT, VT, D_H, D_H), jnp.float32),
            pltpu.VMEM((NSLOT, VT, D_OUT, D_H), jnp.float32),
            pltpu.SemaphoreType.DMA((NSLOT, 3)),
        ],
        compiler_params=pltpu.CompilerParams(
            vmem_limit_bytes=120 * 1024 * 1024,
        ),
    )(
        x.reshape(B, V, 1, D_IN),
        b0.reshape(V, 1, D_H),
        b1.reshape(V, 1, D_H),
        b2.reshape(V, 1, D_OUT),
        W0,
        W1,
        W2,
    )
    return out.reshape(B, V, D_OUT)


# speedup vs baseline: 1.0111x; 1.0016x over previous
import jax
import jax.numpy as jnp
from jax.experimental import pallas as pl
from jax.experimental.pallas import tpu as pltpu

B, V, D_IN, D_H, D_OUT = 1024, 128, 256, 512, 256
VT = 4
NSTEP = V // VT
NSLOT = 4


def _issue(w0_hbm, w1_hbm, w2_hbm, w0_s, w1_s, w2_s, sem, step):
    slot = step % NSLOT
    pltpu.make_async_copy(
        w0_hbm.at[pl.ds(step * VT, VT)], w0_s.at[slot], sem.at[slot, 0]).start()
    pltpu.make_async_copy(
        w1_hbm.at[pl.ds(step * VT, VT)], w1_s.at[slot], sem.at[slot, 1]).start()
    pltpu.make_async_copy(
        w2_hbm.at[pl.ds(step * VT, VT)], w2_s.at[slot], sem.at[slot, 2]).start()


def _wait(w0_s, w1_s, w2_s, sem, step):
    slot = step % NSLOT
    pltpu.make_async_copy(w0_s.at[slot], w0_s.at[slot], sem.at[slot, 0]).wait()
    pltpu.make_async_copy(w1_s.at[slot], w1_s.at[slot], sem.at[slot, 1]).wait()
    pltpu.make_async_copy(w2_s.at[slot], w2_s.at[slot], sem.at[slot, 2]).wait()


def _mlp_kernel(x_ref, b0_ref, b1_ref, b2_ref, w0_hbm, w1_hbm, w2_hbm,
                out_ref, w0_s, w1_s, w2_s, sem):
    v = pl.program_id(0)
    v0 = v * VT

    @pl.when(v == 0)
    def _prologue():
        for s in range(NSLOT):
            _issue(w0_hbm, w1_hbm, w2_hbm, w0_s, w1_s, w2_s, sem, s)

    _wait(w0_s, w1_s, w2_s, sem, v)
    slot = v % NSLOT

    dn = (((1,), (1,)), ((), ()))
    for i in range(VT):
        xv = x_ref[:, i, 0, :]
        h = jax.lax.dot_general(xv, w0_s[slot, i], dn,
                                preferred_element_type=jnp.float32)
        h = jnp.maximum(h + b0_ref[v0 + i], 0.0)
        h = jax.lax.dot_general(h, w1_s[slot, i], dn,
                                preferred_element_type=jnp.float32)
        h = jnp.maximum(h + b1_ref[v0 + i], 0.0)
        o = jax.lax.dot_general(h, w2_s[slot, i], dn,
                                preferred_element_type=jnp.float32)
        out_ref[:, i, 0, :] = o + b2_ref[v0 + i]

    @pl.when(v + NSLOT < NSTEP)
    def _prefetch():
        _issue(w0_hbm, w1_hbm, w2_hbm, w0_s, w1_s, w2_s, sem, v + NSLOT)


def kernel(x, W0, b0, W1, b1, W2, b2):
    out = pl.pallas_call(
        _mlp_kernel,
        grid=(NSTEP,),
        in_specs=[
            pl.BlockSpec((B, VT, 1, D_IN), lambda v: (0, v, 0, 0)),
            pl.BlockSpec((V, 1, D_H), lambda v: (0, 0, 0)),
            pl.BlockSpec((V, 1, D_H), lambda v: (0, 0, 0)),
            pl.BlockSpec((V, 1, D_OUT), lambda v: (0, 0, 0)),
            pl.BlockSpec(memory_space=pltpu.MemorySpace.HBM),
            pl.BlockSpec(memory_space=pltpu.MemorySpace.HBM),
            pl.BlockSpec(memory_space=pltpu.MemorySpace.HBM),
        ],
        out_specs=pl.BlockSpec((B, VT, 1, D_OUT), lambda v: (0, v, 0, 0)),
        out_shape=jax.ShapeDtypeStruct((B, V, 1, D_OUT), jnp.float32),
        scratch_shapes=[
            pltpu.VMEM((NSLOT, VT, D_H, D_IN), jnp.float32),
            pltpu.VMEM((NSLOT, VT, D_H, D_H), jnp.float32),
            pltpu.VMEM((NSLOT, VT, D_OUT, D_H), jnp.float32),
            pltpu.SemaphoreType.DMA((NSLOT, 3)),
        ],
        compiler_params=pltpu.CompilerParams(
            vmem_limit_bytes=120 * 1024 * 1024,
        ),
    )(
        x.reshape(B, V, 1, D_IN),
        b0.reshape(V, 1, D_H),
        b1.reshape(V, 1, D_H),
        b2.reshape(V, 1, D_OUT),
        W0,
        W1,
        W2,
    )
    return out.reshape(B, V, D_OUT)
